# baseline (speedup 1.0000x reference)
import functools

import jax
import jax.numpy as jnp
from jax.experimental import pallas as pl

_SELU_SCALE = 1.0507009873554804934193349852946
_SELU_ALPHA = 1.6732632423543772848170429916717
_NSPLIT = 8


def _body(*refs, block_n):
    a_refs, (x_ref, w_ref, o_ref) = refs[:_NSPLIT], refs[_NSPLIT:]
    x = x_ref[0].astype(jnp.bfloat16)
    w = w_ref[...].astype(jnp.bfloat16)
    for k, a_ref in enumerate(a_refs):
        a = a_ref[0].astype(jnp.bfloat16)
        h = jnp.dot(a, x, preferred_element_type=jnp.float32)
        h = jnp.dot(h.astype(jnp.bfloat16), w, preferred_element_type=jnp.float32)
        h = _SELU_SCALE * jnp.where(h > 0, h, _SELU_ALPHA * (jnp.exp(h) - 1.0))
        o_ref[k * block_n:(k + 1) * block_n, :] = h


@functools.partial(jax.jit, static_argnames=("block_n",))
def _run(Feature_tensor, adj, W, block_n=128):
    n_time, n_node, d = Feature_tensor.shape
    grid = (n_time, n_node // (_NSPLIT * block_n))
    adj_specs = [
        pl.BlockSpec((1, block_n, n_node),
                     functools.partial(lambda k, t, i: (t, _NSPLIT * i + k, 0), k))
        for k in range(_NSPLIT)
    ]
    out = pl.pallas_call(
        functools.partial(_body, block_n=block_n),
        grid=grid,
        in_specs=adj_specs + [
            pl.BlockSpec((1, n_node, d), lambda t, i: (t, 0, 0)),
            pl.BlockSpec((d, d), lambda t, i: (0, 0)),
        ],
        out_specs=pl.BlockSpec((_NSPLIT * block_n, d), lambda t, i: (i, t)),
        out_shape=jax.ShapeDtypeStruct((n_node, n_time * d), jnp.float32),
    )(*([adj] * _NSPLIT), Feature_tensor, W)
    return out.reshape(n_node, n_time, d)


def kernel(Feature_tensor, adj, W):
    return _run(Feature_tensor, adj, W)


# speedup vs baseline: 1.0440x; 1.0440x over previous
import functools

import jax
import jax.numpy as jnp
from jax.experimental import pallas as pl

_SELU_SCALE = 1.0507009873554804934193349852946
_SELU_ALPHA = 1.6732632423543772848170429916717
_NSPLIT = 4


def _body(*refs, block_n):
    a_refs, (x_ref, w_ref, o_ref) = refs[:_NSPLIT], refs[_NSPLIT:]
    x = x_ref[0].astype(jnp.bfloat16)
    w = w_ref[...].astype(jnp.bfloat16)
    for k, a_ref in enumerate(a_refs):
        a = a_ref[0].astype(jnp.bfloat16)
        h = jnp.dot(a, x, preferred_element_type=jnp.float32)
        h = jnp.dot(h.astype(jnp.bfloat16), w, preferred_element_type=jnp.float32)
        h = _SELU_SCALE * jnp.where(h > 0, h, _SELU_ALPHA * (jnp.exp(h) - 1.0))
        o_ref[k * block_n:(k + 1) * block_n, :] = h


@functools.partial(jax.jit, static_argnames=("block_n",))
def _run(Feature_tensor, adj, W, block_n=512):
    n_time, n_node, d = Feature_tensor.shape
    grid = (n_time, n_node // (_NSPLIT * block_n))
    adj_specs = [
        pl.BlockSpec((1, block_n, n_node),
                     functools.partial(lambda k, t, i: (t, _NSPLIT * i + k, 0), k))
        for k in range(_NSPLIT)
    ]
    out = pl.pallas_call(
        functools.partial(_body, block_n=block_n),
        grid=grid,
        in_specs=adj_specs + [
            pl.BlockSpec((1, n_node, d), lambda t, i: (t, 0, 0)),
            pl.BlockSpec((d, d), lambda t, i: (0, 0)),
        ],
        out_specs=pl.BlockSpec((_NSPLIT * block_n, d), lambda t, i: (i, t)),
        out_shape=jax.ShapeDtypeStruct((n_node, n_time * d), jnp.float32),
    )(*([adj] * _NSPLIT), Feature_tensor, W)
    return out.reshape(n_node, n_time, d)


def kernel(Feature_tensor, adj, W):
    return _run(Feature_tensor, adj, W)


# speedup vs baseline: 1.0611x; 1.0164x over previous
import functools

import jax
import jax.numpy as jnp
from jax.experimental import pallas as pl

_SELU_SCALE = 1.0507009873554804934193349852946
_SELU_ALPHA = 1.6732632423543772848170429916717
_NSPLIT = 4


def _body(*refs, block_n):
    a_refs, (x_ref, w_ref, o_ref) = refs[:_NSPLIT], refs[_NSPLIT:]
    x = x_ref[0].astype(jnp.bfloat16)
    w = w_ref[...].astype(jnp.bfloat16)
    for k, a_ref in enumerate(a_refs):
        a = a_ref[0].astype(jnp.bfloat16)
        h = jnp.dot(a, x, preferred_element_type=jnp.float32)
        h = jnp.dot(h.astype(jnp.bfloat16), w, preferred_element_type=jnp.float32)
        h = _SELU_SCALE * jnp.where(h > 0, h, _SELU_ALPHA * (jnp.exp(h) - 1.0))
        o_ref[k * block_n:(k + 1) * block_n, :] = h


@functools.partial(jax.jit, static_argnames=("block_n",))
def _run(Feature_tensor, adj, W, block_n=256):
    n_time, n_node, d = Feature_tensor.shape
    grid = (n_time, n_node // (_NSPLIT * block_n))
    adj_specs = [
        pl.BlockSpec((1, block_n, n_node),
                     functools.partial(lambda k, t, i: (t, _NSPLIT * i + k, 0), k))
        for k in range(_NSPLIT)
    ]
    out = pl.pallas_call(
        functools.partial(_body, block_n=block_n),
        grid=grid,
        in_specs=adj_specs + [
            pl.BlockSpec((1, n_node, d), lambda t, i: (t, 0, 0)),
            pl.BlockSpec((d, d), lambda t, i: (0, 0)),
        ],
        out_specs=pl.BlockSpec((_NSPLIT * block_n, d), lambda t, i: (i, t)),
        out_shape=jax.ShapeDtypeStruct((n_node, n_time * d), jnp.float32),
    )(*([adj] * _NSPLIT), Feature_tensor, W)
    return out.reshape(n_node, n_time, d)


def kernel(Feature_tensor, adj, W):
    return _run(Feature_tensor, adj, W)


# speedup vs baseline: 1.2965x; 1.2218x over previous
import functools

import jax
import jax.numpy as jnp
from jax.experimental import pallas as pl


_SELU_SCALE = 1.0507009873554804934193349852946
_SELU_ALPHA = 1.6732632423543772848170429916717


def _body(*refs, n_time):
    a_refs, (x_ref, w_ref, o_ref) = refs[:n_time], refs[n_time:]
    w = w_ref[...].astype(jnp.bfloat16)
    hs = []
    for t in range(n_time):
        a = a_refs[t][0].astype(jnp.bfloat16)
        x = x_ref[t].astype(jnp.bfloat16)
        h = jnp.dot(a, x, preferred_element_type=jnp.float32)
        h = jnp.dot(h.astype(jnp.bfloat16), w, preferred_element_type=jnp.float32)
        h = _SELU_SCALE * jnp.where(h > 0, h, _SELU_ALPHA * (jnp.exp(h) - 1.0))
        hs.append(h)
    o_ref[...] = jnp.stack(hs, axis=1)


@functools.partial(jax.jit, static_argnames=("block_n",))
def _run(Feature_tensor, adj, W, block_n=128):
    n_time, n_node, d = Feature_tensor.shape
    grid = (n_node // block_n,)
    adj_specs = [
        pl.BlockSpec((1, block_n, n_node),
                     functools.partial(lambda t, i: (t, i, 0), t))
        for t in range(n_time)
    ]
    return pl.pallas_call(
        functools.partial(_body, n_time=n_time),
        grid=grid,
        in_specs=adj_specs + [
            pl.BlockSpec((n_time, n_node, d), lambda i: (0, 0, 0)),
            pl.BlockSpec((d, d), lambda i: (0, 0)),
        ],
        out_specs=pl.BlockSpec((block_n, n_time, d), lambda i: (i, 0, 0)),
        out_shape=jax.ShapeDtypeStruct((n_node, n_time, d), jnp.float32),
    )(*([adj] * n_time), Feature_tensor, W)


def kernel(Feature_tensor, adj, W):
    return _run(Feature_tensor, adj, W)


# speedup vs baseline: 1.3506x; 1.0417x over previous
import functools

import jax
import jax.numpy as jnp
from jax.experimental import pallas as pl


_SELU_SCALE = 1.0507009873554804934193349852946
_SELU_ALPHA = 1.6732632423543772848170429916717
_TSPLIT = 8


def _body(*refs, n_time):
    a_refs, (x_ref, w_ref, o_ref) = refs[:_TSPLIT], refs[_TSPLIT:]
    j = pl.program_id(1)
    w = w_ref[...].astype(jnp.bfloat16)
    hs = []
    for k in range(_TSPLIT):
        a = a_refs[k][0].astype(jnp.bfloat16)
        x = x_ref[j * _TSPLIT + k].astype(jnp.bfloat16)
        h = jnp.dot(a, x, preferred_element_type=jnp.float32)
        h = jnp.dot(h.astype(jnp.bfloat16), w, preferred_element_type=jnp.float32)
        h = _SELU_SCALE * jnp.where(h > 0, h, _SELU_ALPHA * (jnp.exp(h) - 1.0))
        hs.append(h)
    o_ref[...] = jnp.stack(hs, axis=1)


@functools.partial(jax.jit, static_argnames=("block_n",))
def _run(Feature_tensor, adj, W, block_n=256):
    n_time, n_node, d = Feature_tensor.shape
    grid = (n_node // block_n, n_time // _TSPLIT)
    adj_specs = [
        pl.BlockSpec((1, block_n, n_node),
                     functools.partial(lambda k, i, j: (_TSPLIT * j + k, i, 0), k))
        for k in range(_TSPLIT)
    ]
    return pl.pallas_call(
        functools.partial(_body, n_time=n_time),
        grid=grid,
        in_specs=adj_specs + [
            pl.BlockSpec((n_time, n_node, d), lambda i, j: (0, 0, 0)),
            pl.BlockSpec((d, d), lambda i, j: (0, 0)),
        ],
        out_specs=pl.BlockSpec((block_n, _TSPLIT, d), lambda i, j: (i, j, 0)),
        out_shape=jax.ShapeDtypeStruct((n_node, n_time, d), jnp.float32),
    )(*([adj] * _TSPLIT), Feature_tensor, W)


def kernel(Feature_tensor, adj, W):
    return _run(Feature_tensor, adj, W)
